# initial kernel scaffold (unmeasured)
import jax
import jax.numpy as jnp
from jax import lax
from jax.experimental import pallas as pl
from jax.experimental.pallas import tpu as pltpu

N_DEV = 16
MASKS = (1, 3, 4, 8)


def kernel(t, W):
    m, k = t.shape
    kw, n = W.shape

    def body(t_ref, w_ref, out_ref, acc_ref, recv_ref, send_sems, recv_sems):
        my_i = lax.axis_index("i")

        acc_ref[...] = t_ref[...]

        for s, mask in enumerate(MASKS):
            partner = my_i ^ mask
            rdma = pltpu.make_async_remote_copy(
                src_ref=acc_ref,
                dst_ref=recv_ref.at[s],
                send_sem=send_sems.at[s],
                recv_sem=recv_sems.at[s],
                device_id=(partner,),
                device_id_type=pl.DeviceIdType.MESH,
            )
            rdma.start()
            rdma.wait()
            acc_ref[...] = acc_ref[...] + recv_ref[s]

        out_ref[...] = jnp.dot(
            acc_ref[...], w_ref[...], preferred_element_type=jnp.float32
        )

    return pl.pallas_call(
        body,
        out_shape=jax.ShapeDtypeStruct((m, n), jnp.float32),
        in_specs=[
            pl.BlockSpec(memory_space=pltpu.VMEM),
            pl.BlockSpec(memory_space=pltpu.VMEM),
        ],
        out_specs=pl.BlockSpec(memory_space=pltpu.VMEM),
        scratch_shapes=[
            pltpu.VMEM((m, k), jnp.float32),
            pltpu.VMEM((len(MASKS), m, k), jnp.float32),
            pltpu.SemaphoreType.DMA((len(MASKS),)),
            pltpu.SemaphoreType.DMA((len(MASKS),)),
        ],
        compiler_params=pltpu.CompilerParams(collective_id=0),
    )(t, W)


# baseline (device time: 48717 ns/iter reference)
import jax
import jax.numpy as jnp
from jax import lax
from jax.experimental import pallas as pl
from jax.experimental.pallas import tpu as pltpu

N_DEV = 16
MASKS = (1, 3, 4, 8)


def kernel(t, W):
    m, k = t.shape
    kw, n = W.shape

    def body(t_ref, w_ref, out_ref, acc_ref, recv_ref, send_sems, recv_sems):
        my_i = lax.axis_index("i")

        acc_ref[...] = t_ref[...]

        for s, mask in enumerate(MASKS):
            partner = my_i ^ mask
            rdma = pltpu.make_async_remote_copy(
                src_ref=acc_ref,
                dst_ref=recv_ref.at[s],
                send_sem=send_sems.at[s],
                recv_sem=recv_sems.at[s],
                device_id=(partner,),
                device_id_type=pl.DeviceIdType.MESH,
            )
            rdma.start()
            rdma.wait()
            acc_ref[...] = acc_ref[...] + recv_ref[s]

        out_ref[...] = jnp.dot(
            acc_ref[...], w_ref[...], preferred_element_type=jnp.float32
        )

    return pl.pallas_call(
        body,
        out_shape=jax.ShapeDtypeStruct((m, n), jnp.float32),
        in_specs=[
            pl.BlockSpec(memory_space=pltpu.VMEM),
            pl.BlockSpec(memory_space=pltpu.VMEM),
        ],
        out_specs=pl.BlockSpec(memory_space=pltpu.VMEM),
        scratch_shapes=[
            pltpu.VMEM((m, k), jnp.float32),
            pltpu.VMEM((len(MASKS), m, k), jnp.float32),
            pltpu.SemaphoreType.DMA((len(MASKS),)),
            pltpu.SemaphoreType.DMA((len(MASKS),)),
        ],
    )(t, W)


# device time: 31796 ns/iter; 1.5322x vs baseline; 1.5322x over previous
import jax
import jax.numpy as jnp
from jax import lax
from jax.experimental import pallas as pl
from jax.experimental.pallas import tpu as pltpu

N_DEV = 16
MASKS = (1, 3, 4, 8)
CHUNKS = 4


def kernel(t, W):
    m, k = t.shape
    kw, n = W.shape
    ch = m // CHUNKS
    n_steps = len(MASKS)

    def body(t_ref, w_ref, out_ref, acc_ref, recv_ref, send_sems, recv_sems):
        my_i = lax.axis_index("i")

        barrier = pltpu.get_barrier_semaphore()
        for mask in MASKS:
            pl.semaphore_signal(
                barrier,
                inc=1,
                device_id=(my_i ^ mask,),
                device_id_type=pl.DeviceIdType.MESH,
            )
        pl.semaphore_wait(barrier, n_steps)

        acc_ref[...] = t_ref[...]

        def mk(s, c):
            return pltpu.make_async_remote_copy(
                src_ref=acc_ref.at[pl.ds(c * ch, ch), :],
                dst_ref=recv_ref.at[s, pl.ds(c * ch, ch), :],
                send_sem=send_sems.at[s, c],
                recv_sem=recv_sems.at[s, c],
                device_id=(my_i ^ MASKS[s],),
                device_id_type=pl.DeviceIdType.MESH,
            )

        inflight = {}
        for c in range(CHUNKS):
            r = mk(0, c)
            r.start()
            inflight[(0, c)] = r

        for s in range(n_steps):
            for c in range(CHUNKS):
                r = inflight.pop((s, c))
                r.wait_send()
                r.wait_recv()
                rows = pl.ds(c * ch, ch)
                acc_ref[rows, :] = acc_ref[rows, :] + recv_ref[s, rows, :]
                if s + 1 < n_steps:
                    r2 = mk(s + 1, c)
                    r2.start()
                    inflight[(s + 1, c)] = r2

        out_ref[...] = jnp.dot(
            acc_ref[...], w_ref[...], preferred_element_type=jnp.float32
        )

    return pl.pallas_call(
        body,
        out_shape=jax.ShapeDtypeStruct((m, n), jnp.float32),
        in_specs=[
            pl.BlockSpec(memory_space=pltpu.VMEM),
            pl.BlockSpec(memory_space=pltpu.VMEM),
        ],
        out_specs=pl.BlockSpec(memory_space=pltpu.VMEM),
        scratch_shapes=[
            pltpu.VMEM((m, k), jnp.float32),
            pltpu.VMEM((len(MASKS), m, k), jnp.float32),
            pltpu.SemaphoreType.DMA((len(MASKS), CHUNKS)),
            pltpu.SemaphoreType.DMA((len(MASKS), CHUNKS)),
        ],
        compiler_params=pltpu.CompilerParams(collective_id=0),
    )(t, W)


# device time: 31427 ns/iter; 1.5502x vs baseline; 1.0117x over previous
import jax
import jax.numpy as jnp
from jax import lax
from jax.experimental import pallas as pl
from jax.experimental.pallas import tpu as pltpu

N_DEV = 16
MASKS = (1, 3, 4, 8)
CHUNKS = 8


def kernel(t, W):
    m, k = t.shape
    kw, n = W.shape
    ch = m // CHUNKS
    n_steps = len(MASKS)

    def body(t_ref, w_ref, out_ref, acc_ref, recv_ref, send_sems, recv_sems):
        my_i = lax.axis_index("i")

        barrier = pltpu.get_barrier_semaphore()
        for mask in MASKS:
            pl.semaphore_signal(
                barrier,
                inc=1,
                device_id=(my_i ^ mask,),
                device_id_type=pl.DeviceIdType.MESH,
            )
        pl.semaphore_wait(barrier, n_steps)

        acc_ref[...] = t_ref[...]

        def mk(s, c):
            return pltpu.make_async_remote_copy(
                src_ref=acc_ref.at[pl.ds(c * ch, ch), :],
                dst_ref=recv_ref.at[s, pl.ds(c * ch, ch), :],
                send_sem=send_sems.at[s, c],
                recv_sem=recv_sems.at[s, c],
                device_id=(my_i ^ MASKS[s],),
                device_id_type=pl.DeviceIdType.MESH,
            )

        inflight = {}
        for c in range(CHUNKS):
            r = mk(0, c)
            r.start()
            inflight[(0, c)] = r

        for s in range(n_steps):
            for c in range(CHUNKS):
                r = inflight.pop((s, c))
                r.wait_send()
                r.wait_recv()
                rows = pl.ds(c * ch, ch)
                acc_ref[rows, :] = acc_ref[rows, :] + recv_ref[s, rows, :]
                if s + 1 < n_steps:
                    r2 = mk(s + 1, c)
                    r2.start()
                    inflight[(s + 1, c)] = r2
                else:
                    out_ref[rows, :] = jnp.dot(
                        acc_ref[rows, :],
                        w_ref[...],
                        preferred_element_type=jnp.float32,
                    )

    return pl.pallas_call(
        body,
        out_shape=jax.ShapeDtypeStruct((m, n), jnp.float32),
        in_specs=[
            pl.BlockSpec(memory_space=pltpu.VMEM),
            pl.BlockSpec(memory_space=pltpu.VMEM),
        ],
        out_specs=pl.BlockSpec(memory_space=pltpu.VMEM),
        scratch_shapes=[
            pltpu.VMEM((m, k), jnp.float32),
            pltpu.VMEM((len(MASKS), m, k), jnp.float32),
            pltpu.SemaphoreType.DMA((len(MASKS), CHUNKS)),
            pltpu.SemaphoreType.DMA((len(MASKS), CHUNKS)),
        ],
        compiler_params=pltpu.CompilerParams(collective_id=0),
    )(t, W)


# device time: 22446 ns/iter; 2.1704x vs baseline; 1.4001x over previous
import jax
import jax.numpy as jnp
from jax import lax
from jax.experimental import pallas as pl
from jax.experimental.pallas import tpu as pltpu

N_DEV = 16
MASKS = (1, 3, 4, 8)
CHUNKS = 4


def kernel(t, W):
    m, k = t.shape
    kw, n = W.shape
    ch = m // CHUNKS
    n_steps = len(MASKS)

    def body(t_ref, w_ref, out_ref, acc_ref, recv_ref, send_sems, recv_sems):
        my_i = lax.axis_index("i")

        barrier = pltpu.get_barrier_semaphore()
        for mask in MASKS:
            pl.semaphore_signal(
                barrier,
                inc=1,
                device_id=(my_i ^ mask,),
                device_id_type=pl.DeviceIdType.MESH,
            )
        pl.semaphore_wait(barrier, n_steps)

        acc_ref[...] = t_ref[...]

        def mk(s, c):
            return pltpu.make_async_remote_copy(
                src_ref=acc_ref.at[pl.ds(c * ch, ch), :],
                dst_ref=recv_ref.at[s, pl.ds(c * ch, ch), :],
                send_sem=send_sems.at[s, c],
                recv_sem=recv_sems.at[s, c],
                device_id=(my_i ^ MASKS[(s + c) % len(MASKS)],),
                device_id_type=pl.DeviceIdType.MESH,
            )

        inflight = {}
        for c in range(CHUNKS):
            r = mk(0, c)
            r.start()
            inflight[(0, c)] = r

        for s in range(n_steps):
            for c in range(CHUNKS):
                r = inflight.pop((s, c))
                r.wait_send()
                r.wait_recv()
                rows = pl.ds(c * ch, ch)
                acc_ref[rows, :] = acc_ref[rows, :] + recv_ref[s, rows, :]
                if s + 1 < n_steps:
                    r2 = mk(s + 1, c)
                    r2.start()
                    inflight[(s + 1, c)] = r2
                else:
                    out_ref[rows, :] = jnp.dot(
                        acc_ref[rows, :],
                        w_ref[...],
                        preferred_element_type=jnp.float32,
                    )

    return pl.pallas_call(
        body,
        out_shape=jax.ShapeDtypeStruct((m, n), jnp.float32),
        in_specs=[
            pl.BlockSpec(memory_space=pltpu.VMEM),
            pl.BlockSpec(memory_space=pltpu.VMEM),
        ],
        out_specs=pl.BlockSpec(memory_space=pltpu.VMEM),
        scratch_shapes=[
            pltpu.VMEM((m, k), jnp.float32),
            pltpu.VMEM((len(MASKS), m, k), jnp.float32),
            pltpu.SemaphoreType.DMA((len(MASKS), CHUNKS)),
            pltpu.SemaphoreType.DMA((len(MASKS), CHUNKS)),
        ],
        compiler_params=pltpu.CompilerParams(collective_id=0),
    )(t, W)


# device time: 22411 ns/iter; 2.1738x vs baseline; 1.0016x over previous
import jax
import jax.numpy as jnp
from jax import lax
from jax.experimental import pallas as pl
from jax.experimental.pallas import tpu as pltpu

N_DEV = 16
MASKS = (1, 3, 4, 8)
CHUNKS = 4


def kernel(t, W):
    m, k = t.shape
    kw, n = W.shape
    ch = m // CHUNKS
    n_steps = len(MASKS)

    def body(t_ref, w_ref, out_ref, acc_ref, recv_ref, send_sems, recv_sems):
        my_i = lax.axis_index("i")

        barrier = pltpu.get_barrier_semaphore()
        for mask in MASKS:
            pl.semaphore_signal(
                barrier,
                inc=1,
                device_id=(my_i ^ mask,),
                device_id_type=pl.DeviceIdType.MESH,
            )
        pl.semaphore_wait(barrier, n_steps)

        def mk(s, c):
            src = t_ref if s == 0 else acc_ref
            return pltpu.make_async_remote_copy(
                src_ref=src.at[pl.ds(c * ch, ch), :],
                dst_ref=recv_ref.at[s, pl.ds(c * ch, ch), :],
                send_sem=send_sems.at[s, c],
                recv_sem=recv_sems.at[s, c],
                device_id=(my_i ^ MASKS[(s + c) % len(MASKS)],),
                device_id_type=pl.DeviceIdType.MESH,
            )

        inflight = {}
        for c in range(CHUNKS):
            r = mk(0, c)
            r.start()
            inflight[(0, c)] = r

        for s in range(n_steps):
            for c in range(CHUNKS):
                r = inflight.pop((s, c))
                r.wait_send()
                r.wait_recv()
                rows = pl.ds(c * ch, ch)
                base = t_ref if s == 0 else acc_ref
                acc_ref[rows, :] = base[rows, :] + recv_ref[s, rows, :]
                if s + 1 < n_steps:
                    r2 = mk(s + 1, c)
                    r2.start()
                    inflight[(s + 1, c)] = r2
                else:
                    out_ref[rows, :] = jnp.dot(
                        acc_ref[rows, :],
                        w_ref[...],
                        preferred_element_type=jnp.float32,
                    )

    return pl.pallas_call(
        body,
        out_shape=jax.ShapeDtypeStruct((m, n), jnp.float32),
        in_specs=[
            pl.BlockSpec(memory_space=pltpu.VMEM),
            pl.BlockSpec(memory_space=pltpu.VMEM),
        ],
        out_specs=pl.BlockSpec(memory_space=pltpu.VMEM),
        scratch_shapes=[
            pltpu.VMEM((m, k), jnp.float32),
            pltpu.VMEM((len(MASKS), m, k), jnp.float32),
            pltpu.SemaphoreType.DMA((len(MASKS), CHUNKS)),
            pltpu.SemaphoreType.DMA((len(MASKS), CHUNKS)),
        ],
        compiler_params=pltpu.CompilerParams(collective_id=0),
    )(t, W)


# device time: 22001 ns/iter; 2.2143x vs baseline; 1.0186x over previous
import jax
import jax.numpy as jnp
from jax import lax
from jax.experimental import pallas as pl
from jax.experimental.pallas import tpu as pltpu

N_DEV = 16
MASKS = (1, 3, 4, 8)
CHUNKS = 8


def kernel(t, W):
    m, k = t.shape
    kw, n = W.shape
    ch = m // CHUNKS
    n_steps = len(MASKS)

    def body(t_ref, w_ref, out_ref, acc_ref, recv_ref, send_sems, recv_sems):
        my_i = lax.axis_index("i")

        barrier = pltpu.get_barrier_semaphore()
        for mask in MASKS:
            pl.semaphore_signal(
                barrier,
                inc=1,
                device_id=(my_i ^ mask,),
                device_id_type=pl.DeviceIdType.MESH,
            )
        pl.semaphore_wait(barrier, n_steps)

        def mk(s, c):
            src = t_ref if s == 0 else acc_ref
            return pltpu.make_async_remote_copy(
                src_ref=src.at[pl.ds(c * ch, ch), :],
                dst_ref=recv_ref.at[s, pl.ds(c * ch, ch), :],
                send_sem=send_sems.at[s, c],
                recv_sem=recv_sems.at[s, c],
                device_id=(my_i ^ MASKS[(s + c) % len(MASKS)],),
                device_id_type=pl.DeviceIdType.MESH,
            )

        inflight = {}
        for c in range(CHUNKS):
            r = mk(0, c)
            r.start()
            inflight[(0, c)] = r

        for s in range(n_steps):
            for c in range(CHUNKS):
                r = inflight.pop((s, c))
                r.wait_send()
                r.wait_recv()
                rows = pl.ds(c * ch, ch)
                base = t_ref if s == 0 else acc_ref
                acc_ref[rows, :] = base[rows, :] + recv_ref[s, rows, :]
                if s + 1 < n_steps:
                    r2 = mk(s + 1, c)
                    r2.start()
                    inflight[(s + 1, c)] = r2
                else:
                    out_ref[rows, :] = jnp.dot(
                        acc_ref[rows, :],
                        w_ref[...],
                        preferred_element_type=jnp.float32,
                    )

    return pl.pallas_call(
        body,
        out_shape=jax.ShapeDtypeStruct((m, n), jnp.float32),
        in_specs=[
            pl.BlockSpec(memory_space=pltpu.VMEM),
            pl.BlockSpec(memory_space=pltpu.VMEM),
        ],
        out_specs=pl.BlockSpec(memory_space=pltpu.VMEM),
        scratch_shapes=[
            pltpu.VMEM((m, k), jnp.float32),
            pltpu.VMEM((len(MASKS), m, k), jnp.float32),
            pltpu.SemaphoreType.DMA((len(MASKS), CHUNKS)),
            pltpu.SemaphoreType.DMA((len(MASKS), CHUNKS)),
        ],
        compiler_params=pltpu.CompilerParams(collective_id=0),
    )(t, W)
